# baseline (speedup 1.0000x reference)
import functools

import jax
import jax.numpy as jnp
from jax import lax
from jax.experimental import pallas as pl
from jax.experimental.pallas import tpu as pltpu
from jax.experimental.pallas import tpu_sc as plsc

SRC = 32768
TGT = 8192
D = 128

NC = 2
NS = 16
NW = NC * NS
SRC_PER_W = SRC // NW
OUT_PER_W = TGT // NW

ROWS_BLK = 2048
N_BLK = SRC // ROWS_BLK

_HI = jax.lax.Precision.HIGHEST


def _logits_body(z_ref, w1t_ref, b1_ref, w2pt_ref, b2_ref, out_ref):
    h = lax.dot_general(z_ref[...], w1t_ref[...], (((1,), (0,)), ((), ())),
                        preferred_element_type=jnp.float32)
    h = jnp.tanh(h + b1_ref[...])
    o = lax.dot_general(h, w2pt_ref[...], (((1,), (0,)), ((), ())),
                        preferred_element_type=jnp.float32)
    out_ref[...] = o[:, 0:1] + b2_ref[0, 0]


def _compute_logits(z, w1t, b1r, w2pt, b2r):
    return pl.pallas_call(
        _logits_body,
        grid=(N_BLK,),
        in_specs=[
            pl.BlockSpec((ROWS_BLK, D), lambda i: (i, 0)),
            pl.BlockSpec((D, D), lambda i: (0, 0)),
            pl.BlockSpec((1, D), lambda i: (0, 0)),
            pl.BlockSpec((D, D), lambda i: (0, 0)),
            pl.BlockSpec((1, 1), lambda i: (0, 0)),
        ],
        out_specs=pl.BlockSpec((ROWS_BLK, 1), lambda i: (i, 0)),
        out_shape=jax.ShapeDtypeStruct((SRC, 1), jnp.float32),
    )(z, w1t, b1r, w2pt, b2r)


def _excl_cumsum_rowmajor(a_f32, tri_lane, tri_row):
    incl = lax.dot_general(a_f32, tri_lane, (((1,), (0,)), ((), ())),
                           precision=_HI, preferred_element_type=jnp.float32)
    row_tot = incl[:, D - 1:D]
    row_incl = lax.dot_general(tri_row, row_tot, (((1,), (0,)), ((), ())),
                               precision=_HI, preferred_element_type=jnp.float32)
    return incl - a_f32 + (row_incl - row_tot)


def _select_body(lg_ref, sel_ref, pos_ref):
    x = lg_ref[...]
    s = lax.bitcast_convert_type(x, jnp.int32)
    key = s ^ (lax.shift_right_arithmetic(s, 31) & jnp.int32(0x7FFFFFFF))

    k = jnp.int32(TGT)
    c_pos = jnp.sum((key >= 0).astype(jnp.int32))
    base0 = jnp.where(c_pos >= k, jnp.int32(0), jnp.int32(-2147483648))

    def step(i, base):
        b = jnp.int32(30) - i
        t = base + lax.shift_left(jnp.int32(1), b)
        c = jnp.sum((key >= t).astype(jnp.int32))
        return jnp.where(c >= k, t, base)

    vk = lax.fori_loop(0, 31, step, base0)

    gt = key > vk
    tie = key == vk
    cg = jnp.sum(gt.astype(jnp.int32))
    r = k - cg

    rows = lax.broadcasted_iota(jnp.int32, (D, D), 0)
    cols = lax.broadcasted_iota(jnp.int32, (D, D), 1)
    tri_lane = (rows <= cols).astype(jnp.float32)
    r2 = lax.broadcasted_iota(jnp.int32, (SRC // D, SRC // D), 0)
    c2 = lax.broadcasted_iota(jnp.int32, (SRC // D, SRC // D), 1)
    tri_row = (c2 <= r2).astype(jnp.float32)

    tie_rank = _excl_cumsum_rowmajor(tie.astype(jnp.float32), tri_lane, tri_row)
    sel = gt | (tie & (tie_rank.astype(jnp.int32) < r))
    pos = _excl_cumsum_rowmajor(sel.astype(jnp.float32), tri_lane, tri_row)

    sel_ref[...] = sel.astype(jnp.int32)
    pos_ref[...] = pos.astype(jnp.int32)


def _select_rank(logits2d):
    return pl.pallas_call(
        _select_body,
        out_shape=(
            jax.ShapeDtypeStruct((SRC // D, D), jnp.int32),
            jax.ShapeDtypeStruct((SRC // D, D), jnp.int32),
        ),
    )(logits2d)


def _compact_body(sel_hbm, pos_hbm, fold_hbm, idx_hbm,
                  sel_v, pos_v, fold_v, val_v, dest_v, sem):
    wid = lax.axis_index("s") * NC + lax.axis_index("c")
    base = wid * SRC_PER_W
    pltpu.sync_copy(sel_hbm.at[pl.ds(base, SRC_PER_W)], sel_v)
    pltpu.sync_copy(pos_hbm.at[pl.ds(base, SRC_PER_W)], pos_v)
    pltpu.sync_copy(fold_hbm, fold_v)
    fold = fold_v[...]
    lane = lax.iota(jnp.int32, 16)
    dummy = jnp.full((16,), TGT, jnp.int32) + wid
    for j in range(SRC_PER_W // 16):
        s = sel_v[pl.ds(j * 16, 16)]
        p = pos_v[pl.ds(j * 16, 16)]
        src = jnp.full((16,), base + j * 16, jnp.int32) + lane
        val = jnp.clip(src + fold, 0, SRC - 1)
        dest = jnp.where(s > 0, p, dummy)
        row = j // (D // 16)
        col = (j % (D // 16)) * 16
        val_v[row, pl.ds(col, 16)] = val
        dest_v[row, pl.ds(col, 16)] = dest
    for rrow in range(SRC_PER_W // D):
        pltpu.async_copy(val_v.at[rrow], idx_hbm.at[dest_v.at[rrow]], sem).wait()


def _gather_body(idx_hbm, z_hbm, out_hbm, idx_v, rows_v, sem):
    wid = lax.axis_index("s") * NC + lax.axis_index("c")
    base = wid * OUT_PER_W
    for i in range(OUT_PER_W // D):
        pltpu.sync_copy(idx_hbm.at[pl.ds(base + i * D, D)], idx_v.at[i])
    for i in range(OUT_PER_W // D):
        pltpu.async_copy(z_hbm.at[idx_v.at[i]],
                         rows_v.at[pl.ds(i * D, D)], sem).wait()
    pltpu.sync_copy(rows_v, out_hbm.at[pl.ds(base, OUT_PER_W)])


@functools.lru_cache(maxsize=1)
def _sc_kernels():
    mesh = plsc.VectorSubcoreMesh(core_axis_name="c", subcore_axis_name="s")
    compact = pl.kernel(
        _compact_body,
        out_type=jax.ShapeDtypeStruct((TGT + NW,), jnp.int32),
        mesh=mesh,
        scratch_types=[
            pltpu.VMEM((SRC_PER_W,), jnp.int32),
            pltpu.VMEM((SRC_PER_W,), jnp.int32),
            pltpu.VMEM((16,), jnp.int32),
            pltpu.VMEM((SRC_PER_W // D, D), jnp.int32),
            pltpu.VMEM((SRC_PER_W // D, D), jnp.int32),
            pltpu.SemaphoreType.DMA,
        ],
    )
    gather = pl.kernel(
        _gather_body,
        out_type=jax.ShapeDtypeStruct((TGT, D), jnp.float32),
        mesh=mesh,
        scratch_types=[
            pltpu.VMEM((OUT_PER_W // D, D), jnp.int32),
            pltpu.VMEM((OUT_PER_W, D), jnp.float32),
            pltpu.SemaphoreType.DMA,
        ],
    )
    return compact, gather


def kernel(z_struct, W1, b1, W2, b2, target_length):
    w2pt = jnp.zeros((D, D), jnp.float32).at[:, 0].set(W2[0])
    logits = _compute_logits(z_struct, W1.T, b1.reshape(1, D),
                             w2pt, b2.reshape(1, 1))
    sel, pos = _select_rank(logits.reshape(SRC // D, D))
    fold = jnp.full((16,), jnp.asarray(target_length, jnp.int32) - TGT, jnp.int32)
    compact, gather = _sc_kernels()
    idx_full = compact(sel.reshape(SRC), pos.reshape(SRC), fold)
    return gather(idx_full, z_struct)


# speedup vs baseline: 23.9541x; 23.9541x over previous
import functools

import jax
import jax.numpy as jnp
from jax import lax
from jax.experimental import pallas as pl
from jax.experimental.pallas import tpu as pltpu
from jax.experimental.pallas import tpu_sc as plsc

SRC = 32768
TGT = 8192
D = 128

NC = 2
NS = 16
NW = NC * NS
SRC_PER_W = SRC // NW
OUT_PER_W = TGT // NW

ROWS_BLK = 2048
N_BLK = SRC // ROWS_BLK

_HI = jax.lax.Precision.HIGHEST


def _logits_body(z_ref, w1t_ref, b1_ref, w2pt_ref, b2_ref, out_ref):
    h = lax.dot_general(z_ref[...], w1t_ref[...], (((1,), (0,)), ((), ())),
                        preferred_element_type=jnp.float32)
    h = jnp.tanh(h + b1_ref[...])
    o = lax.dot_general(h, w2pt_ref[...], (((1,), (0,)), ((), ())),
                        preferred_element_type=jnp.float32)
    out_ref[...] = o[:, 0:1] + b2_ref[0, 0]


def _compute_logits(z, w1t, b1r, w2pt, b2r):
    return pl.pallas_call(
        _logits_body,
        grid=(N_BLK,),
        in_specs=[
            pl.BlockSpec((ROWS_BLK, D), lambda i: (i, 0)),
            pl.BlockSpec((D, D), lambda i: (0, 0)),
            pl.BlockSpec((1, D), lambda i: (0, 0)),
            pl.BlockSpec((D, D), lambda i: (0, 0)),
            pl.BlockSpec((1, 1), lambda i: (0, 0)),
        ],
        out_specs=pl.BlockSpec((ROWS_BLK, 1), lambda i: (i, 0)),
        out_shape=jax.ShapeDtypeStruct((SRC, 1), jnp.float32),
    )(z, w1t, b1r, w2pt, b2r)


def _excl_cumsum_rowmajor(a_f32, tri_lane, tri_row):
    incl = lax.dot_general(a_f32, tri_lane, (((1,), (0,)), ((), ())),
                           precision=_HI, preferred_element_type=jnp.float32)
    row_tot = incl[:, D - 1:D]
    row_incl = lax.dot_general(tri_row, row_tot, (((1,), (0,)), ((), ())),
                               precision=_HI, preferred_element_type=jnp.float32)
    return incl - a_f32 + (row_incl - row_tot)


def _select_body(lg_ref, sel_ref, pos_ref):
    x = lg_ref[...]
    s = lax.bitcast_convert_type(x, jnp.int32)
    key = s ^ (lax.shift_right_arithmetic(s, 31) & jnp.int32(0x7FFFFFFF))

    k = jnp.int32(TGT)
    c_pos = jnp.sum((key >= 0).astype(jnp.int32))
    base0 = jnp.where(c_pos >= k, jnp.int32(0), jnp.int32(-2147483648))

    def step(i, base):
        b = jnp.int32(30) - i
        t = base + lax.shift_left(jnp.int32(1), b)
        c = jnp.sum((key >= t).astype(jnp.int32))
        return jnp.where(c >= k, t, base)

    vk = lax.fori_loop(0, 31, step, base0)

    gt = key > vk
    tie = key == vk
    cg = jnp.sum(gt.astype(jnp.int32))
    r = k - cg

    rows = lax.broadcasted_iota(jnp.int32, (D, D), 0)
    cols = lax.broadcasted_iota(jnp.int32, (D, D), 1)
    tri_lane = (rows <= cols).astype(jnp.float32)
    r2 = lax.broadcasted_iota(jnp.int32, (SRC // D, SRC // D), 0)
    c2 = lax.broadcasted_iota(jnp.int32, (SRC // D, SRC // D), 1)
    tri_row = (c2 <= r2).astype(jnp.float32)

    tie_rank = _excl_cumsum_rowmajor(tie.astype(jnp.float32), tri_lane, tri_row)
    sel = gt | (tie & (tie_rank.astype(jnp.int32) < r))
    pos = _excl_cumsum_rowmajor(sel.astype(jnp.float32), tri_lane, tri_row)

    sel_ref[...] = sel.astype(jnp.int32)
    pos_ref[...] = pos.astype(jnp.int32)


def _select_rank(logits2d):
    return pl.pallas_call(
        _select_body,
        out_shape=(
            jax.ShapeDtypeStruct((SRC // D, D), jnp.int32),
            jax.ShapeDtypeStruct((SRC // D, D), jnp.int32),
        ),
    )(logits2d)


def _compact_body(sel_hbm, pos_hbm, fold_hbm, idx_hbm,
                  sel_v, pos_v, fold_v, val_v, dest_v, sem):
    wid = lax.axis_index("s") * NC + lax.axis_index("c")
    base = wid * SRC_PER_W
    pltpu.sync_copy(sel_hbm.at[pl.ds(base, SRC_PER_W)], sel_v)
    pltpu.sync_copy(pos_hbm.at[pl.ds(base, SRC_PER_W)], pos_v)
    pltpu.sync_copy(fold_hbm, fold_v)
    fold = fold_v[...]
    lane = lax.iota(jnp.int32, 16)
    for j in range(SRC_PER_W // 16):
        s = sel_v[pl.ds(j * 16, 16)]
        p = pos_v[pl.ds(j * 16, 16)]
        src = jnp.full((16,), base + j * 16, jnp.int32) + lane
        val = jnp.clip(src + fold, 0, SRC - 1)
        dest = jnp.where(s > 0, p, src + TGT)
        row = j // (D // 16)
        col = (j % (D // 16)) * 16
        val_v[row, pl.ds(col, 16)] = val
        dest_v[row, pl.ds(col, 16)] = dest
    for rrow in range(SRC_PER_W // D):
        pltpu.async_copy(val_v.at[rrow], idx_hbm.at[dest_v.at[rrow]], sem).wait()


def _gather_body(idx_hbm, z_hbm, out_hbm, idx_v, rows_v, sem):
    wid = lax.axis_index("s") * NC + lax.axis_index("c")
    base = wid * OUT_PER_W
    for i in range(OUT_PER_W // D):
        pltpu.sync_copy(idx_hbm.at[pl.ds(base + i * D, D)], idx_v.at[i])
    for i in range(OUT_PER_W // D):
        pltpu.async_copy(z_hbm.at[idx_v.at[i]],
                         rows_v.at[pl.ds(i * D, D)], sem).wait()
    pltpu.sync_copy(rows_v, out_hbm.at[pl.ds(base, OUT_PER_W)])


@functools.lru_cache(maxsize=1)
def _sc_kernels():
    mesh = plsc.VectorSubcoreMesh(core_axis_name="c", subcore_axis_name="s")
    compact = pl.kernel(
        _compact_body,
        out_type=jax.ShapeDtypeStruct((TGT + SRC,), jnp.int32),
        mesh=mesh,
        scratch_types=[
            pltpu.VMEM((SRC_PER_W,), jnp.int32),
            pltpu.VMEM((SRC_PER_W,), jnp.int32),
            pltpu.VMEM((16,), jnp.int32),
            pltpu.VMEM((SRC_PER_W // D, D), jnp.int32),
            pltpu.VMEM((SRC_PER_W // D, D), jnp.int32),
            pltpu.SemaphoreType.DMA,
        ],
    )
    gather = pl.kernel(
        _gather_body,
        out_type=jax.ShapeDtypeStruct((TGT, D), jnp.float32),
        mesh=mesh,
        scratch_types=[
            pltpu.VMEM((OUT_PER_W // D, D), jnp.int32),
            pltpu.VMEM((OUT_PER_W, D), jnp.float32),
            pltpu.SemaphoreType.DMA,
        ],
    )
    return compact, gather


def kernel(z_struct, W1, b1, W2, b2, target_length):
    w2pt = jnp.zeros((D, D), jnp.float32).at[:, 0].set(W2[0])
    logits = _compute_logits(z_struct, W1.T, b1.reshape(1, D),
                             w2pt, b2.reshape(1, 1))
    sel, pos = _select_rank(logits.reshape(SRC // D, D))
    fold = jnp.full((16,), jnp.asarray(target_length, jnp.int32) - TGT, jnp.int32)
    compact, gather = _sc_kernels()
    idx_full = compact(sel.reshape(SRC), pos.reshape(SRC), fold)
    return gather(idx_full, z_struct)


# speedup vs baseline: 24.0232x; 1.0029x over previous
import functools

import jax
import jax.numpy as jnp
from jax import lax
from jax.experimental import pallas as pl
from jax.experimental.pallas import tpu as pltpu
from jax.experimental.pallas import tpu_sc as plsc

SRC = 32768
TGT = 8192
D = 128

NC = 2
NS = 16
NW = NC * NS
SRC_PER_W = SRC // NW
OUT_PER_W = TGT // NW

ROWS_BLK = 2048
N_BLK = SRC // ROWS_BLK

_HI = jax.lax.Precision.HIGHEST


def _logits_body(z_ref, w1t_ref, b1_ref, w2pt_ref, b2_ref, out_ref):
    h = lax.dot_general(z_ref[...], w1t_ref[...], (((1,), (0,)), ((), ())),
                        preferred_element_type=jnp.float32)
    h = jnp.tanh(h + b1_ref[...])
    o = lax.dot_general(h, w2pt_ref[...], (((1,), (0,)), ((), ())),
                        preferred_element_type=jnp.float32)
    out_ref[...] = o[:, 0:1] + b2_ref[0, 0]


def _compute_logits(z, w1t, b1r, w2pt, b2r):
    return pl.pallas_call(
        _logits_body,
        grid=(N_BLK,),
        in_specs=[
            pl.BlockSpec((ROWS_BLK, D), lambda i: (i, 0)),
            pl.BlockSpec((D, D), lambda i: (0, 0)),
            pl.BlockSpec((1, D), lambda i: (0, 0)),
            pl.BlockSpec((D, D), lambda i: (0, 0)),
            pl.BlockSpec((1, 1), lambda i: (0, 0)),
        ],
        out_specs=pl.BlockSpec((ROWS_BLK, 1), lambda i: (i, 0)),
        out_shape=jax.ShapeDtypeStruct((SRC, 1), jnp.float32),
    )(z, w1t, b1r, w2pt, b2r)


def _excl_cumsum_rowmajor(a_f32, tri_lane, tri_row):
    incl = lax.dot_general(a_f32, tri_lane, (((1,), (0,)), ((), ())),
                           precision=_HI, preferred_element_type=jnp.float32)
    row_tot = incl[:, D - 1:D]
    row_incl = lax.dot_general(tri_row, row_tot, (((1,), (0,)), ((), ())),
                               precision=_HI, preferred_element_type=jnp.float32)
    return incl - a_f32 + (row_incl - row_tot)


def _select_body(lg_ref, sel_ref, pos_ref):
    x = lg_ref[...]
    s = lax.bitcast_convert_type(x, jnp.int32)
    key = s ^ (lax.shift_right_arithmetic(s, 31) & jnp.int32(0x7FFFFFFF))

    k = jnp.int32(TGT)
    c_pos = jnp.sum((key >= 0).astype(jnp.int32))
    base0 = jnp.where(c_pos >= k, jnp.int32(0), jnp.int32(-2147483648))

    def step(i, base):
        b = jnp.int32(30) - i
        t = base + lax.shift_left(jnp.int32(1), b)
        c = jnp.sum((key >= t).astype(jnp.int32))
        return jnp.where(c >= k, t, base)

    vk = lax.fori_loop(0, 31, step, base0)

    gt = key > vk
    tie = key == vk
    cg = jnp.sum(gt.astype(jnp.int32))
    r = k - cg

    rows = lax.broadcasted_iota(jnp.int32, (D, D), 0)
    cols = lax.broadcasted_iota(jnp.int32, (D, D), 1)
    tri_lane = (rows <= cols).astype(jnp.float32)
    r2 = lax.broadcasted_iota(jnp.int32, (SRC // D, SRC // D), 0)
    c2 = lax.broadcasted_iota(jnp.int32, (SRC // D, SRC // D), 1)
    tri_row = (c2 <= r2).astype(jnp.float32)

    tie_rank = _excl_cumsum_rowmajor(tie.astype(jnp.float32), tri_lane, tri_row)
    sel = gt | (tie & (tie_rank.astype(jnp.int32) < r))
    pos = _excl_cumsum_rowmajor(sel.astype(jnp.float32), tri_lane, tri_row)

    sel_ref[...] = sel.astype(jnp.int32)
    pos_ref[...] = pos.astype(jnp.int32)


def _select_rank(logits2d):
    return pl.pallas_call(
        _select_body,
        out_shape=(
            jax.ShapeDtypeStruct((SRC // D, D), jnp.int32),
            jax.ShapeDtypeStruct((SRC // D, D), jnp.int32),
        ),
    )(logits2d)


def _compact_body(sel_hbm, pos_hbm, fold_hbm, idx_hbm,
                  sel_v, pos_v, fold_v, val_v, dest_v, sem):
    wid = lax.axis_index("s") * NC + lax.axis_index("c")
    base = wid * SRC_PER_W
    pltpu.sync_copy(sel_hbm.at[pl.ds(base, SRC_PER_W)], sel_v)
    pltpu.sync_copy(pos_hbm.at[pl.ds(base, SRC_PER_W)], pos_v)
    pltpu.sync_copy(fold_hbm, fold_v)
    fold = fold_v[...]
    lane = lax.iota(jnp.int32, 16)
    for j in range(SRC_PER_W // 16):
        s = sel_v[pl.ds(j * 16, 16)]
        p = pos_v[pl.ds(j * 16, 16)]
        src = jnp.full((16,), base + j * 16, jnp.int32) + lane
        val = jnp.clip(src + fold, 0, SRC - 1)
        dest = jnp.where(s > 0, p, src + TGT)
        row = j // (D // 16)
        col = (j % (D // 16)) * 16
        val_v[row, pl.ds(col, 16)] = val
        dest_v[row, pl.ds(col, 16)] = dest
    copies = [pltpu.async_copy(val_v.at[rrow], idx_hbm.at[dest_v.at[rrow]], sem)
              for rrow in range(SRC_PER_W // D)]
    for c in copies:
        c.wait()


def _gather_body(idx_hbm, z_hbm, out_hbm, idx_v, rows_v, sem):
    wid = lax.axis_index("s") * NC + lax.axis_index("c")
    base = wid * OUT_PER_W
    for i in range(OUT_PER_W // D):
        pltpu.sync_copy(idx_hbm.at[pl.ds(base + i * D, D)], idx_v.at[i])
    for i in range(OUT_PER_W // D):
        pltpu.async_copy(z_hbm.at[idx_v.at[i]],
                         rows_v.at[pl.ds(i * D, D)], sem).wait()
    pltpu.sync_copy(rows_v, out_hbm.at[pl.ds(base, OUT_PER_W)])


@functools.lru_cache(maxsize=1)
def _sc_kernels():
    mesh = plsc.VectorSubcoreMesh(core_axis_name="c", subcore_axis_name="s")
    compact = pl.kernel(
        _compact_body,
        out_type=jax.ShapeDtypeStruct((TGT + SRC,), jnp.int32),
        mesh=mesh,
        scratch_types=[
            pltpu.VMEM((SRC_PER_W,), jnp.int32),
            pltpu.VMEM((SRC_PER_W,), jnp.int32),
            pltpu.VMEM((16,), jnp.int32),
            pltpu.VMEM((SRC_PER_W // D, D), jnp.int32),
            pltpu.VMEM((SRC_PER_W // D, D), jnp.int32),
            pltpu.SemaphoreType.DMA,
        ],
    )
    gather = pl.kernel(
        _gather_body,
        out_type=jax.ShapeDtypeStruct((TGT, D), jnp.float32),
        mesh=mesh,
        scratch_types=[
            pltpu.VMEM((OUT_PER_W // D, D), jnp.int32),
            pltpu.VMEM((OUT_PER_W, D), jnp.float32),
            pltpu.SemaphoreType.DMA,
        ],
    )
    return compact, gather


def kernel(z_struct, W1, b1, W2, b2, target_length):
    w2pt = jnp.zeros((D, D), jnp.float32).at[:, 0].set(W2[0])
    logits = _compute_logits(z_struct, W1.T, b1.reshape(1, D),
                             w2pt, b2.reshape(1, 1))
    sel, pos = _select_rank(logits.reshape(SRC // D, D))
    fold = jnp.full((16,), jnp.asarray(target_length, jnp.int32) - TGT, jnp.int32)
    compact, gather = _sc_kernels()
    idx_full = compact(sel.reshape(SRC), pos.reshape(SRC), fold)
    return gather(idx_full, z_struct)


# speedup vs baseline: 51.0800x; 2.1263x over previous
import functools

import jax
import jax.numpy as jnp
from jax import lax
from jax.experimental import pallas as pl
from jax.experimental.pallas import tpu as pltpu
from jax.experimental.pallas import tpu_sc as plsc

SRC = 32768
TGT = 8192
D = 128

NC = 2
NS = 16
NW = NC * NS
SRC_PER_W = SRC // NW
OUT_PER_W = TGT // NW

ROWS_BLK = 2048
N_BLK = SRC // ROWS_BLK

_HI = jax.lax.Precision.HIGHEST


def _logits_body(z_ref, w1t_ref, b1_ref, w2pt_ref, b2_ref, out_ref):
    h = lax.dot_general(z_ref[...], w1t_ref[...], (((1,), (0,)), ((), ())),
                        preferred_element_type=jnp.float32)
    h = jnp.tanh(h + b1_ref[...])
    o = lax.dot_general(h, w2pt_ref[...], (((1,), (0,)), ((), ())),
                        preferred_element_type=jnp.float32)
    out_ref[...] = o[:, 0:1] + b2_ref[0, 0]


def _compute_logits(z, w1t, b1r, w2pt, b2r):
    return pl.pallas_call(
        _logits_body,
        grid=(N_BLK,),
        in_specs=[
            pl.BlockSpec((ROWS_BLK, D), lambda i: (i, 0)),
            pl.BlockSpec((D, D), lambda i: (0, 0)),
            pl.BlockSpec((1, D), lambda i: (0, 0)),
            pl.BlockSpec((D, D), lambda i: (0, 0)),
            pl.BlockSpec((1, 1), lambda i: (0, 0)),
        ],
        out_specs=pl.BlockSpec((ROWS_BLK, 1), lambda i: (i, 0)),
        out_shape=jax.ShapeDtypeStruct((SRC, 1), jnp.float32),
    )(z, w1t, b1r, w2pt, b2r)


def _excl_cumsum_rowmajor(a_f32, tri_lane, tri_row):
    incl = lax.dot_general(a_f32, tri_lane, (((1,), (0,)), ((), ())),
                           precision=_HI, preferred_element_type=jnp.float32)
    row_tot = incl[:, D - 1:D]
    row_incl = lax.dot_general(tri_row, row_tot, (((1,), (0,)), ((), ())),
                               precision=_HI, preferred_element_type=jnp.float32)
    return incl - a_f32 + (row_incl - row_tot)


def _select_body(lg_ref, sel_ref, pos_ref):
    x = lg_ref[...]
    s = lax.bitcast_convert_type(x, jnp.int32)
    key = s ^ (lax.shift_right_arithmetic(s, 31) & jnp.int32(0x7FFFFFFF))

    k = jnp.int32(TGT)
    c_pos = jnp.sum((key >= 0).astype(jnp.int32))
    base0 = jnp.where(c_pos >= k, jnp.int32(0), jnp.int32(-2147483648))

    def step(i, base):
        b = jnp.int32(30) - i
        t = base + lax.shift_left(jnp.int32(1), b)
        c = jnp.sum((key >= t).astype(jnp.int32))
        return jnp.where(c >= k, t, base)

    vk = lax.fori_loop(0, 31, step, base0)

    gt = key > vk
    tie = key == vk
    cg = jnp.sum(gt.astype(jnp.int32))
    r = k - cg

    rows = lax.broadcasted_iota(jnp.int32, (D, D), 0)
    cols = lax.broadcasted_iota(jnp.int32, (D, D), 1)
    tri_lane = (rows <= cols).astype(jnp.float32)
    r2 = lax.broadcasted_iota(jnp.int32, (SRC // D, SRC // D), 0)
    c2 = lax.broadcasted_iota(jnp.int32, (SRC // D, SRC // D), 1)
    tri_row = (c2 <= r2).astype(jnp.float32)

    tie_rank = _excl_cumsum_rowmajor(tie.astype(jnp.float32), tri_lane, tri_row)
    sel = gt | (tie & (tie_rank.astype(jnp.int32) < r))
    pos = _excl_cumsum_rowmajor(sel.astype(jnp.float32), tri_lane, tri_row)

    sel_ref[...] = sel.astype(jnp.int32)
    pos_ref[...] = pos.astype(jnp.int32)


def _select_rank(logits2d):
    return pl.pallas_call(
        _select_body,
        out_shape=(
            jax.ShapeDtypeStruct((SRC // D, D), jnp.int32),
            jax.ShapeDtypeStruct((SRC // D, D), jnp.int32),
        ),
    )(logits2d)


def _compact_body(sel_hbm, pos_hbm, fold_hbm, part_hbm,
                  sel_v, pos_v, fold_v, val_v, dest_v, zero_v, shared, sem):
    cid = lax.axis_index("c")
    sid = lax.axis_index("s")
    wid = sid * NC + cid
    base = wid * SRC_PER_W
    zeros16 = jnp.zeros((16,), jnp.int32)
    for j in range(TGT // NS // 16):
        zero_v[pl.ds(j * 16, 16)] = zeros16
    pltpu.sync_copy(zero_v, shared.at[pl.ds(sid * (TGT // NS), TGT // NS)])
    pltpu.sync_copy(sel_hbm.at[pl.ds(base, SRC_PER_W)], sel_v)
    pltpu.sync_copy(pos_hbm.at[pl.ds(base, SRC_PER_W)], pos_v)
    pltpu.sync_copy(fold_hbm, fold_v)
    fold = fold_v[...]
    lane = lax.iota(jnp.int32, 16)
    for j in range(SRC_PER_W // 16):
        s = sel_v[pl.ds(j * 16, 16)]
        p = pos_v[pl.ds(j * 16, 16)]
        src = jnp.full((16,), base + j * 16, jnp.int32) + lane
        val = jnp.clip(src + fold, 0, SRC - 1) + 1
        dest = jnp.where(s > 0, p, src + TGT)
        row = j // (D // 16)
        col = (j % (D // 16)) * 16
        val_v[row, pl.ds(col, 16)] = val
        dest_v[row, pl.ds(col, 16)] = dest
    plsc.subcore_barrier()
    for rrow in range(SRC_PER_W // D):
        pltpu.sync_copy(val_v.at[rrow], shared.at[dest_v.at[rrow]])
    plsc.subcore_barrier()
    pltpu.sync_copy(shared.at[pl.ds(sid * (TGT // NS), TGT // NS)],
                    part_hbm.at[cid, pl.ds(sid * (TGT // NS), TGT // NS)])


def _gather_body(part_hbm, z_hbm, out_hbm, a_v, b_v, idx_v, rows_v, sem):
    wid = lax.axis_index("s") * NC + lax.axis_index("c")
    base = wid * OUT_PER_W
    for i in range(OUT_PER_W // D):
        pltpu.sync_copy(part_hbm.at[0, pl.ds(base + i * D, D)], a_v.at[i])
        pltpu.sync_copy(part_hbm.at[1, pl.ds(base + i * D, D)], b_v.at[i])
    for j in range(OUT_PER_W // 16):
        row = j // (D // 16)
        col = (j % (D // 16)) * 16
        a = a_v[row, pl.ds(col, 16)]
        b = b_v[row, pl.ds(col, 16)]
        idx_v[row, pl.ds(col, 16)] = jnp.maximum(a, b) - 1
    for i in range(OUT_PER_W // D):
        pltpu.async_copy(z_hbm.at[idx_v.at[i]],
                         rows_v.at[pl.ds(i * D, D)], sem).wait()
    pltpu.sync_copy(rows_v, out_hbm.at[pl.ds(base, OUT_PER_W)])


@functools.lru_cache(maxsize=1)
def _sc_kernels():
    mesh = plsc.VectorSubcoreMesh(core_axis_name="c", subcore_axis_name="s")
    compact = pl.kernel(
        _compact_body,
        out_type=jax.ShapeDtypeStruct((NC, TGT), jnp.int32),
        mesh=mesh,
        scratch_types=[
            pltpu.VMEM((SRC_PER_W,), jnp.int32),
            pltpu.VMEM((SRC_PER_W,), jnp.int32),
            pltpu.VMEM((16,), jnp.int32),
            pltpu.VMEM((SRC_PER_W // D, D), jnp.int32),
            pltpu.VMEM((SRC_PER_W // D, D), jnp.int32),
            pltpu.VMEM((TGT // NS,), jnp.int32),
            pltpu.VMEM_SHARED((TGT + SRC,), jnp.int32),
            pltpu.SemaphoreType.DMA,
        ],
    )
    gather = pl.kernel(
        _gather_body,
        out_type=jax.ShapeDtypeStruct((TGT, D), jnp.float32),
        mesh=mesh,
        scratch_types=[
            pltpu.VMEM((OUT_PER_W // D, D), jnp.int32),
            pltpu.VMEM((OUT_PER_W // D, D), jnp.int32),
            pltpu.VMEM((OUT_PER_W // D, D), jnp.int32),
            pltpu.VMEM((OUT_PER_W, D), jnp.float32),
            pltpu.SemaphoreType.DMA,
        ],
    )
    return compact, gather


def kernel(z_struct, W1, b1, W2, b2, target_length):
    w2pt = jnp.zeros((D, D), jnp.float32).at[:, 0].set(W2[0])
    logits = _compute_logits(z_struct, W1.T, b1.reshape(1, D),
                             w2pt, b2.reshape(1, 1))
    sel, pos = _select_rank(logits.reshape(SRC // D, D))
    fold = jnp.full((16,), jnp.asarray(target_length, jnp.int32) - TGT, jnp.int32)
    compact, gather = _sc_kernels()
    partials = compact(sel.reshape(SRC), pos.reshape(SRC), fold)
    return gather(partials, z_struct)


# speedup vs baseline: 53.1069x; 1.0397x over previous
import functools

import jax
import jax.numpy as jnp
from jax import lax
from jax.experimental import pallas as pl
from jax.experimental.pallas import tpu as pltpu
from jax.experimental.pallas import tpu_sc as plsc

SRC = 32768
TGT = 8192
D = 128

NC = 2
NS = 16
NW = NC * NS
SRC_PER_W = SRC // NW
OUT_PER_W = TGT // NW

ROWS_BLK = 2048
N_BLK = SRC // ROWS_BLK

_HI = jax.lax.Precision.HIGHEST


def _logits_body(z_ref, w1t_ref, b1_ref, w2pt_ref, b2_ref, out_ref):
    h = lax.dot_general(z_ref[...], w1t_ref[...], (((1,), (0,)), ((), ())),
                        preferred_element_type=jnp.float32)
    h = jnp.tanh(h + b1_ref[...])
    o = lax.dot_general(h, w2pt_ref[...], (((1,), (0,)), ((), ())),
                        preferred_element_type=jnp.float32)
    out_ref[...] = o[:, 0:1] + b2_ref[0, 0]


def _compute_logits(z, w1t, b1r, w2pt, b2r):
    return pl.pallas_call(
        _logits_body,
        grid=(N_BLK,),
        in_specs=[
            pl.BlockSpec((ROWS_BLK, D), lambda i: (i, 0)),
            pl.BlockSpec((D, D), lambda i: (0, 0)),
            pl.BlockSpec((1, D), lambda i: (0, 0)),
            pl.BlockSpec((D, D), lambda i: (0, 0)),
            pl.BlockSpec((1, 1), lambda i: (0, 0)),
        ],
        out_specs=pl.BlockSpec((ROWS_BLK, 1), lambda i: (i, 0)),
        out_shape=jax.ShapeDtypeStruct((SRC, 1), jnp.float32),
    )(z, w1t, b1r, w2pt, b2r)


def _excl_cumsum_rowmajor(a_f32, tri_lane, tri_row):
    incl = lax.dot_general(a_f32, tri_lane, (((1,), (0,)), ((), ())),
                           precision=_HI, preferred_element_type=jnp.float32)
    row_tot = incl[:, D - 1:D]
    row_incl = lax.dot_general(tri_row, row_tot, (((1,), (0,)), ((), ())),
                               precision=_HI, preferred_element_type=jnp.float32)
    return incl - a_f32 + (row_incl - row_tot)


def _select_body(lg_ref, sel_ref, pos_ref):
    x = lg_ref[...]
    s = lax.bitcast_convert_type(x, jnp.int32)
    key = s ^ (lax.shift_right_arithmetic(s, 31) & jnp.int32(0x7FFFFFFF))

    k = jnp.int32(TGT)
    c_pos = jnp.sum((key >= 0).astype(jnp.int32))
    base0 = jnp.where(c_pos >= k, jnp.int32(0), jnp.int32(-2147483648))

    def step(i, base):
        b = jnp.int32(30) - i
        t = base + lax.shift_left(jnp.int32(1), b)
        c = jnp.sum((key >= t).astype(jnp.int32))
        return jnp.where(c >= k, t, base)

    vk = lax.fori_loop(0, 31, step, base0)

    gt = key > vk
    tie = key == vk
    cg = jnp.sum(gt.astype(jnp.int32))
    r = k - cg

    rows = lax.broadcasted_iota(jnp.int32, (D, D), 0)
    cols = lax.broadcasted_iota(jnp.int32, (D, D), 1)
    tri_lane = (rows <= cols).astype(jnp.float32)
    r2 = lax.broadcasted_iota(jnp.int32, (SRC // D, SRC // D), 0)
    c2 = lax.broadcasted_iota(jnp.int32, (SRC // D, SRC // D), 1)
    tri_row = (c2 <= r2).astype(jnp.float32)

    tie_rank = _excl_cumsum_rowmajor(tie.astype(jnp.float32), tri_lane, tri_row)
    sel = gt | (tie & (tie_rank.astype(jnp.int32) < r))
    pos = _excl_cumsum_rowmajor(sel.astype(jnp.float32), tri_lane, tri_row)

    sel_ref[...] = sel.astype(jnp.int32)
    pos_ref[...] = pos.astype(jnp.int32)


def _select_rank(logits2d):
    return pl.pallas_call(
        _select_body,
        out_shape=(
            jax.ShapeDtypeStruct((SRC // D, D), jnp.int32),
            jax.ShapeDtypeStruct((SRC // D, D), jnp.int32),
        ),
    )(logits2d)


SPW = SRC // NS
OPW = TGT // NS


def _select_gather_body(sel_hbm, pos_hbm, fold_hbm, z_hbm, out_hbm,
                        sel_v, pos_v, fold_v, val_v, dest_v, idx_v, rows_v,
                        shared, sem):
    sid = lax.axis_index("s")
    base = sid * SPW
    pltpu.sync_copy(sel_hbm.at[pl.ds(base, SPW)], sel_v)
    pltpu.sync_copy(pos_hbm.at[pl.ds(base, SPW)], pos_v)
    pltpu.sync_copy(fold_hbm, fold_v)
    fold = fold_v[...]
    lane = lax.iota(jnp.int32, 16)
    for j in range(SPW // 16):
        s = sel_v[pl.ds(j * 16, 16)]
        p = pos_v[pl.ds(j * 16, 16)]
        src = jnp.full((16,), base + j * 16, jnp.int32) + lane
        val = jnp.clip(src + fold, 0, SRC - 1)
        dest = jnp.where(s > 0, p, src + TGT)
        row = j // (D // 16)
        col = (j % (D // 16)) * 16
        val_v[row, pl.ds(col, 16)] = val
        dest_v[row, pl.ds(col, 16)] = dest
    for rrow in range(SPW // D):
        pltpu.sync_copy(val_v.at[rrow], shared.at[dest_v.at[rrow]])
    plsc.subcore_barrier()
    for i in range(OPW // D):
        pltpu.sync_copy(shared.at[pl.ds(sid * OPW + i * D, D)], idx_v.at[i])
    for i in range(OPW // D):
        pltpu.async_copy(z_hbm.at[idx_v.at[i]],
                         rows_v.at[pl.ds(i * D, D)], sem).wait()
    pltpu.sync_copy(rows_v, out_hbm.at[pl.ds(sid * OPW, OPW)])


@functools.lru_cache(maxsize=1)
def _sc_kernels():
    mesh = plsc.VectorSubcoreMesh(core_axis_name="c", subcore_axis_name="s",
                                  num_cores=1)
    select_gather = pl.kernel(
        _select_gather_body,
        out_type=jax.ShapeDtypeStruct((TGT, D), jnp.float32),
        mesh=mesh,
        scratch_types=[
            pltpu.VMEM((SPW,), jnp.int32),
            pltpu.VMEM((SPW,), jnp.int32),
            pltpu.VMEM((16,), jnp.int32),
            pltpu.VMEM((SPW // D, D), jnp.int32),
            pltpu.VMEM((SPW // D, D), jnp.int32),
            pltpu.VMEM((OPW // D, D), jnp.int32),
            pltpu.VMEM((OPW, D), jnp.float32),
            pltpu.VMEM_SHARED((TGT + SRC,), jnp.int32),
            pltpu.SemaphoreType.DMA,
        ],
    )
    return select_gather


def kernel(z_struct, W1, b1, W2, b2, target_length):
    w2pt = jnp.zeros((D, D), jnp.float32).at[:, 0].set(W2[0])
    logits = _compute_logits(z_struct, W1.T, b1.reshape(1, D),
                             w2pt, b2.reshape(1, 1))
    sel, pos = _select_rank(logits.reshape(SRC // D, D))
    fold = jnp.full((16,), jnp.asarray(target_length, jnp.int32) - TGT, jnp.int32)
    select_gather = _sc_kernels()
    return select_gather(sel.reshape(SRC), pos.reshape(SRC), fold, z_struct)


# speedup vs baseline: 56.4276x; 1.0625x over previous
import functools

import jax
import jax.numpy as jnp
from jax import lax
from jax.experimental import pallas as pl
from jax.experimental.pallas import tpu as pltpu
from jax.experimental.pallas import tpu_sc as plsc

SRC = 32768
TGT = 8192
D = 128

NC = 2
NS = 16
NW = NC * NS
SRC_PER_W = SRC // NW
OUT_PER_W = TGT // NW

ROWS_BLK = 2048
N_BLK = SRC // ROWS_BLK

_HI = jax.lax.Precision.HIGHEST


def _logits_body(z_ref, w1_ref, b1_ref, w2_ref, b2_ref, out_ref):
    h = lax.dot_general(z_ref[...], w1_ref[...], (((1,), (1,)), ((), ())),
                        preferred_element_type=jnp.float32)
    h = jnp.tanh(h + b1_ref[...])
    rows = lax.broadcasted_iota(jnp.int32, (D, D), 0)
    w2p = jnp.where(rows == 0, w2_ref[...], jnp.float32(0.0))
    o = lax.dot_general(h, w2p, (((1,), (1,)), ((), ())),
                        preferred_element_type=jnp.float32)
    out_ref[...] = o[:, 0:1] + b2_ref[0, 0]


def _compute_logits(z, w1, b1r, w2r, b2r):
    return pl.pallas_call(
        _logits_body,
        grid=(N_BLK,),
        in_specs=[
            pl.BlockSpec((ROWS_BLK, D), lambda i: (i, 0)),
            pl.BlockSpec((D, D), lambda i: (0, 0)),
            pl.BlockSpec((1, D), lambda i: (0, 0)),
            pl.BlockSpec((1, D), lambda i: (0, 0)),
            pl.BlockSpec((1, 1), lambda i: (0, 0)),
        ],
        out_specs=pl.BlockSpec((ROWS_BLK, 1), lambda i: (i, 0)),
        out_shape=jax.ShapeDtypeStruct((SRC, 1), jnp.float32),
    )(z, w1, b1r, w2r, b2r)


def _excl_cumsum_rowmajor(a_f32, tri_lane, tri_row):
    incl = lax.dot_general(a_f32, tri_lane, (((1,), (0,)), ((), ())),
                           precision=_HI, preferred_element_type=jnp.float32)
    row_tot = incl[:, D - 1:D]
    row_incl = lax.dot_general(tri_row, row_tot, (((1,), (0,)), ((), ())),
                               precision=_HI, preferred_element_type=jnp.float32)
    return incl - a_f32 + (row_incl - row_tot)


def _select_body(lg_ref, sel_ref, pos_ref):
    x = lg_ref[...]
    s = lax.bitcast_convert_type(x, jnp.int32)
    key = s ^ (lax.shift_right_arithmetic(s, 31) & jnp.int32(0x7FFFFFFF))

    k = jnp.int32(TGT)
    c_pos = jnp.sum((key >= 0).astype(jnp.int32))
    base0 = jnp.where(c_pos >= k, jnp.int32(0), jnp.int32(-2147483648))

    def step(i, base):
        b = jnp.int32(30) - i
        t = base + lax.shift_left(jnp.int32(1), b)
        c = jnp.sum((key >= t).astype(jnp.int32))
        return jnp.where(c >= k, t, base)

    vk = lax.fori_loop(0, 31, step, base0)

    gt = key > vk
    tie = key == vk
    cg = jnp.sum(gt.astype(jnp.int32))
    r = k - cg

    rows = lax.broadcasted_iota(jnp.int32, (D, D), 0)
    cols = lax.broadcasted_iota(jnp.int32, (D, D), 1)
    tri_lane = (rows <= cols).astype(jnp.float32)
    r2 = lax.broadcasted_iota(jnp.int32, (SRC // D, SRC // D), 0)
    c2 = lax.broadcasted_iota(jnp.int32, (SRC // D, SRC // D), 1)
    tri_row = (c2 <= r2).astype(jnp.float32)

    tie_rank = _excl_cumsum_rowmajor(tie.astype(jnp.float32), tri_lane, tri_row)
    sel = gt | (tie & (tie_rank.astype(jnp.int32) < r))
    pos = _excl_cumsum_rowmajor(sel.astype(jnp.float32), tri_lane, tri_row)

    sel_ref[...] = sel.astype(jnp.int32)
    pos_ref[...] = pos.astype(jnp.int32)


def _select_rank(logits2d):
    return pl.pallas_call(
        _select_body,
        out_shape=(
            jax.ShapeDtypeStruct((SRC // D, D), jnp.int32),
            jax.ShapeDtypeStruct((SRC // D, D), jnp.int32),
        ),
    )(logits2d)


SPW = SRC // NS
OPW = TGT // NS


def _select_gather_body(sel_hbm, pos_hbm, fold_hbm, z_hbm, out_hbm,
                        sel_v, pos_v, fold_v, val_v, dest_v, idx_v, rows_v,
                        shared, sem):
    sid = lax.axis_index("s")
    base = sid * SPW
    pltpu.sync_copy(sel_hbm.at[pl.ds(base, SPW)], sel_v)
    pltpu.sync_copy(pos_hbm.at[pl.ds(base, SPW)], pos_v)
    pltpu.sync_copy(fold_hbm, fold_v)
    fold = fold_v[...]
    lane = lax.iota(jnp.int32, 16)
    for j in range(SPW // 16):
        s = sel_v[pl.ds(j * 16, 16)]
        p = pos_v[pl.ds(j * 16, 16)]
        src = jnp.full((16,), base + j * 16, jnp.int32) + lane
        val = jnp.clip(src + fold, 0, SRC - 1)
        dest = jnp.where(s > 0, p, src + TGT)
        row = j // (D // 16)
        col = (j % (D // 16)) * 16
        val_v[row, pl.ds(col, 16)] = val
        dest_v[row, pl.ds(col, 16)] = dest
    for rrow in range(SPW // D):
        pltpu.sync_copy(val_v.at[rrow], shared.at[dest_v.at[rrow]])
    plsc.subcore_barrier()
    for i in range(OPW // D):
        pltpu.sync_copy(shared.at[pl.ds(sid * OPW + i * D, D)], idx_v.at[i])
    for i in range(OPW // D):
        pltpu.async_copy(z_hbm.at[idx_v.at[i]],
                         rows_v.at[pl.ds(i * D, D)], sem).wait()
    pltpu.sync_copy(rows_v, out_hbm.at[pl.ds(sid * OPW, OPW)])


@functools.lru_cache(maxsize=1)
def _sc_kernels():
    mesh = plsc.VectorSubcoreMesh(core_axis_name="c", subcore_axis_name="s",
                                  num_cores=1)
    select_gather = pl.kernel(
        _select_gather_body,
        out_type=jax.ShapeDtypeStruct((TGT, D), jnp.float32),
        mesh=mesh,
        scratch_types=[
            pltpu.VMEM((SPW,), jnp.int32),
            pltpu.VMEM((SPW,), jnp.int32),
            pltpu.VMEM((16,), jnp.int32),
            pltpu.VMEM((SPW // D, D), jnp.int32),
            pltpu.VMEM((SPW // D, D), jnp.int32),
            pltpu.VMEM((OPW // D, D), jnp.int32),
            pltpu.VMEM((OPW, D), jnp.float32),
            pltpu.VMEM_SHARED((TGT + SRC,), jnp.int32),
            pltpu.SemaphoreType.DMA,
        ],
    )
    return select_gather


def kernel(z_struct, W1, b1, W2, b2, target_length):
    logits = _compute_logits(z_struct, W1, b1.reshape(1, D),
                             W2.reshape(1, D), b2.reshape(1, 1))
    sel, pos = _select_rank(logits.reshape(SRC // D, D))
    fold = jnp.full((16,), jnp.asarray(target_length, jnp.int32) - TGT, jnp.int32)
    select_gather = _sc_kernels()
    return select_gather(sel.reshape(SRC), pos.reshape(SRC), fold, z_struct)
